# Initial kernel scaffold; baseline (speedup 1.0000x reference)
import jax
import jax.numpy as jnp
from jax.experimental import pallas as pl


def kernel(x, var_embed, Wq, bq, Wk, bk, Ws, bs, Wc1, bc1, Wc2, bc2, Wcn, bcn, Wg, bg, Wf, bf, ln_w, ln_b):
    raise NotImplementedError("write your pallas kernel here")


# speedup vs baseline: 17.1939x; 17.1939x over previous
import functools

import jax
import jax.numpy as jnp
from jax.experimental import pallas as pl
from jax.experimental.pallas import tpu as pltpu

_TOPK = 8
_TEMP = 1.0
_NEG = -3e38


def _dot_t(a, b):
    return jax.lax.dot_general(
        a, b, (((1,), (1,)), ((), ())), preferred_element_type=jnp.float32)


def _router_kernel(ve_ref, wq_ref, bq_ref, wk_ref, bk_ref, s_ref):
    ve = ve_ref[...]
    q = _dot_t(ve, wq_ref[...]) + bq_ref[...]
    k = _dot_t(ve, wk_ref[...]) + bk_ref[...]
    sim = _dot_t(q, k)
    n = sim.shape[0]
    r = jax.lax.broadcasted_iota(jnp.int32, sim.shape, 0)
    c = jax.lax.broadcasted_iota(jnp.int32, sim.shape, 1)
    sim = jnp.where(r == c, -1e9, sim)
    s = sim
    mask = jnp.zeros(sim.shape, jnp.bool_)
    for _ in range(_TOPK):
        m = jnp.max(s, axis=-1, keepdims=True)
        first = jnp.min(jnp.where(s == m, c, n), axis=-1, keepdims=True)
        sel = c == first
        mask = jnp.logical_or(mask, sel)
        s = jnp.where(sel, _NEG, s)
    mx = jnp.max(jnp.where(mask, sim, _NEG), axis=-1, keepdims=True)
    p = jnp.where(mask, jnp.exp((sim - mx) / _TEMP), 0.0)
    s_ref[...] = p / jnp.sum(p, axis=-1, keepdims=True)


def _gelu_exact(v):
    return 0.5 * v * (1.0 + jax.lax.erf(v * 0.7071067811865476))


def _center_kernel(x_ref, ws_ref, bs_ref, wc1_ref, bc1_ref, wc2_ref, bc2_ref,
                   wcn_ref, bcn_ref, wg2_ref, bg_ref, center_ref, cgate_ref):
    xb = x_ref[0]
    scores = _dot_t(ws_ref[...], xb) + bs_ref[0, 0]
    m = jnp.max(scores, axis=-1, keepdims=True)
    e = jnp.exp(scores - m)
    aw = e / jnp.sum(e, axis=-1, keepdims=True)
    center_raw = jnp.dot(aw, xb, preferred_element_type=jnp.float32)
    h = _gelu_exact(_dot_t(center_raw, wc1_ref[...]) + bc1_ref[...])
    h = _gelu_exact(_dot_t(h, wc2_ref[...]) + bc2_ref[...])
    cv = _dot_t(h, wcn_ref[...]) + bcn_ref[...]
    center_ref[0] = cv
    cgate_ref[0] = _dot_t(cv, wg2_ref[...]) + bg_ref[...]


def _main_kernel(x_ref, s_ref, wg1_ref, wf_ref, bf_ref, lnw_ref, lnb_ref,
                 center_ref, cgate_ref, out_ref):
    xb = x_ref[0]
    ring = _dot_t(xb, s_ref[...])
    gl = _dot_t(ring, wg1_ref[...]) + cgate_ref[0]
    g = jax.nn.sigmoid(gl)
    fused = g * ring + (1.0 - g) * center_ref[0]
    y = _dot_t(fused, wf_ref[...]) + bf_ref[...]
    z = y + xb
    mu = jnp.mean(z, axis=-1, keepdims=True)
    zc = z - mu
    var = jnp.mean(zc * zc, axis=-1, keepdims=True)
    out_ref[0] = zc * jax.lax.rsqrt(var + 1e-5) * lnw_ref[...] + lnb_ref[...]


@jax.jit
def kernel(x, var_embed, Wq, bq, Wk, bk, Ws, bs, Wc1, bc1, Wc2, bc2, Wcn, bcn,
           Wg, bg, Wf, bf, ln_w, ln_b):
    B, L, N = x.shape
    H = var_embed.shape[-1]
    D = Wc1.shape[0]
    f32 = jnp.float32

    ve = var_embed.reshape(N, H)
    row = lambda v: v.reshape(1, -1)
    Wg1 = Wg[:, :N]
    Wg2 = Wg[:, N:]

    S = pl.pallas_call(
        _router_kernel,
        out_shape=jax.ShapeDtypeStruct((N, N), f32),
    )(ve, Wq, row(bq), Wk, row(bk))

    center, cgate = pl.pallas_call(
        _center_kernel,
        grid=(B,),
        in_specs=[
            pl.BlockSpec((1, L, N), lambda b: (b, 0, 0)),
            pl.BlockSpec((1, N), lambda b: (0, 0)),
            pl.BlockSpec((1, 1), lambda b: (0, 0)),
            pl.BlockSpec((D, N), lambda b: (0, 0)),
            pl.BlockSpec((1, D), lambda b: (0, 0)),
            pl.BlockSpec((D, D), lambda b: (0, 0)),
            pl.BlockSpec((1, D), lambda b: (0, 0)),
            pl.BlockSpec((N, D), lambda b: (0, 0)),
            pl.BlockSpec((1, N), lambda b: (0, 0)),
            pl.BlockSpec((N, N), lambda b: (0, 0)),
            pl.BlockSpec((1, N), lambda b: (0, 0)),
        ],
        out_specs=[
            pl.BlockSpec((1, 1, N), lambda b: (b, 0, 0)),
            pl.BlockSpec((1, 1, N), lambda b: (b, 0, 0)),
        ],
        out_shape=[
            jax.ShapeDtypeStruct((B, 1, N), f32),
            jax.ShapeDtypeStruct((B, 1, N), f32),
        ],
    )(x, Ws, bs.reshape(1, 1), Wc1, row(bc1), Wc2, row(bc2), Wcn, row(bcn),
      Wg2, row(bg))

    TL = 256
    out = pl.pallas_call(
        _main_kernel,
        grid=(B, L // TL),
        in_specs=[
            pl.BlockSpec((1, TL, N), lambda b, t: (b, t, 0)),
            pl.BlockSpec((N, N), lambda b, t: (0, 0)),
            pl.BlockSpec((N, N), lambda b, t: (0, 0)),
            pl.BlockSpec((N, N), lambda b, t: (0, 0)),
            pl.BlockSpec((1, N), lambda b, t: (0, 0)),
            pl.BlockSpec((1, N), lambda b, t: (0, 0)),
            pl.BlockSpec((1, N), lambda b, t: (0, 0)),
            pl.BlockSpec((1, 1, N), lambda b, t: (b, 0, 0)),
            pl.BlockSpec((1, 1, N), lambda b, t: (b, 0, 0)),
        ],
        out_specs=pl.BlockSpec((1, TL, N), lambda b, t: (b, t, 0)),
        out_shape=jax.ShapeDtypeStruct((B, L, N), f32),
    )(x, S, Wg1, Wf, row(bf), row(ln_w), row(ln_b), center, cgate)
    return out


# speedup vs baseline: 24.0245x; 1.3973x over previous
import jax
import jax.numpy as jnp
from jax.experimental import pallas as pl
from jax.experimental.pallas import tpu as pltpu

_TOPK = 8
_TEMP = 1.0
_NEG = -3e38


def _dot_t(a, b):
    return jax.lax.dot_general(
        a, b, (((1,), (1,)), ((), ())), preferred_element_type=jnp.float32)


def _gelu_exact(v):
    return 0.5 * v * (1.0 + jax.lax.erf(v * 0.7071067811865476))


def _fused_kernel(x_ref, ve_ref, wq_ref, bq_ref, wk_ref, bk_ref, ws_ref,
                  bs_ref, wc1_ref, bc1_ref, wc2_ref, bc2_ref, wcn_ref,
                  bcn_ref, wg2_ref, bg_ref, wg1b_ref, wfb_ref, bf_ref,
                  lnw_ref, lnb_ref, out_ref, s_ref):
    b = pl.program_id(0)

    @pl.when(b == 0)
    def _router():
        ve = ve_ref[...]
        q = _dot_t(ve, wq_ref[...]) + bq_ref[...]
        k = _dot_t(ve, wk_ref[...]) + bk_ref[...]
        sim = _dot_t(q, k)
        n = sim.shape[0]
        r = jax.lax.broadcasted_iota(jnp.int32, sim.shape, 0)
        c = jax.lax.broadcasted_iota(jnp.int32, sim.shape, 1)
        sim = jnp.where(r == c, -1e9, sim)
        s = sim
        mask = jnp.zeros(sim.shape, jnp.bool_)
        for _ in range(_TOPK):
            m = jnp.max(s, axis=-1, keepdims=True)
            first = jnp.min(jnp.where(s == m, c, n), axis=-1, keepdims=True)
            sel = c == first
            mask = jnp.logical_or(mask, sel)
            s = jnp.where(sel, _NEG, s)
        mx = jnp.max(jnp.where(mask, sim, _NEG), axis=-1, keepdims=True)
        p = jnp.where(mask, jnp.exp((sim - mx) / _TEMP), 0.0)
        s_ref[...] = (p / jnp.sum(p, axis=-1, keepdims=True)).astype(
            jnp.bfloat16)

    xb = x_ref[0]

    scores = _dot_t(ws_ref[...], xb) + bs_ref[0, 0]
    m = jnp.max(scores, axis=-1, keepdims=True)
    e = jnp.exp(scores - m)
    aw = e / jnp.sum(e, axis=-1, keepdims=True)
    center_raw = jnp.dot(aw, xb, preferred_element_type=jnp.float32)
    h = _gelu_exact(_dot_t(center_raw, wc1_ref[...]) + bc1_ref[...])
    h = _gelu_exact(_dot_t(h, wc2_ref[...]) + bc2_ref[...])
    cv = _dot_t(h, wcn_ref[...]) + bcn_ref[...]
    cgate = _dot_t(cv, wg2_ref[...]) + bg_ref[...]

    ring = _dot_t(xb.astype(jnp.bfloat16), s_ref[...])
    gl = _dot_t(ring.astype(jnp.bfloat16), wg1b_ref[...]) + cgate
    g = jax.nn.sigmoid(gl)
    fused = g * ring + (1.0 - g) * cv
    y = _dot_t(fused.astype(jnp.bfloat16), wfb_ref[...]) + bf_ref[...]
    z = y + xb
    mu = jnp.mean(z, axis=-1, keepdims=True)
    zc = z - mu
    var = jnp.mean(zc * zc, axis=-1, keepdims=True)
    out_ref[0] = zc * jax.lax.rsqrt(var + 1e-5) * lnw_ref[...] + lnb_ref[...]


@jax.jit
def kernel(x, var_embed, Wq, bq, Wk, bk, Ws, bs, Wc1, bc1, Wc2, bc2, Wcn, bcn,
           Wg, bg, Wf, bf, ln_w, ln_b):
    B, L, N = x.shape
    H = var_embed.shape[-1]
    D = Wc1.shape[0]
    f32 = jnp.float32
    bf16 = jnp.bfloat16

    ve = var_embed.reshape(N, H)
    row = lambda v: v.reshape(1, -1)
    Wg2 = Wg[:, N:]
    Wg1b = Wg[:, :N].astype(bf16)
    Wfb = Wf.astype(bf16)

    const = lambda *shape: pl.BlockSpec(shape, lambda b: (0,) * len(shape))
    out = pl.pallas_call(
        _fused_kernel,
        grid=(B,),
        in_specs=[
            pl.BlockSpec((1, L, N), lambda b: (b, 0, 0)),
            const(N, H), const(H, H), const(1, H), const(H, H), const(1, H),
            const(1, N), const(1, 1),
            const(D, N), const(1, D), const(D, D), const(1, D),
            const(N, D), const(1, N),
            const(N, N), const(1, N),
            const(N, N), const(N, N), const(1, N), const(1, N), const(1, N),
        ],
        out_specs=pl.BlockSpec((1, L, N), lambda b: (b, 0, 0)),
        out_shape=jax.ShapeDtypeStruct((B, L, N), f32),
        scratch_shapes=[pltpu.VMEM((N, N), bf16)],
    )(x, ve, Wq, row(bq), Wk, row(bk), Ws, bs.reshape(1, 1),
      Wc1, row(bc1), Wc2, row(bc2), Wcn, row(bcn),
      Wg2, row(bg), Wg1b, Wfb, row(bf), row(ln_w), row(ln_b))
    return out


# speedup vs baseline: 24.3938x; 1.0154x over previous
import jax
import jax.numpy as jnp
from jax.experimental import pallas as pl
from jax.experimental.pallas import tpu as pltpu

_TOPK = 8
_TEMP = 1.0
_NEG = -3e38


def _dot_t(a, b):
    return jax.lax.dot_general(
        a, b, (((1,), (1,)), ((), ())), preferred_element_type=jnp.float32)


def _gelu_exact(v):
    return 0.5 * v * (1.0 + jax.lax.erf(v * 0.7071067811865476))


def _fused_kernel(x_ref, ve_ref, wq_ref, bq_ref, wk_ref, bk_ref, ws_ref,
                  bs_ref, wc1_ref, bc1_ref, wc2_ref, bc2_ref, wcn_ref,
                  bcn_ref, wg2_ref, bg_ref, wg1_ref, wfb_ref, bf_ref,
                  lnw_ref, lnb_ref, out_ref, s_ref, wgs_ref):
    b = pl.program_id(0)

    @pl.when(b == 0)
    def _router():
        ve = ve_ref[...]
        q = _dot_t(ve, wq_ref[...]) + bq_ref[...]
        k = _dot_t(ve, wk_ref[...]) + bk_ref[...]
        sim = _dot_t(q, k)
        n = sim.shape[0]
        r = jax.lax.broadcasted_iota(jnp.int32, sim.shape, 0)
        c = jax.lax.broadcasted_iota(jnp.int32, sim.shape, 1)
        sim = jnp.where(r == c, -1e9, sim)
        s = sim
        mask = jnp.zeros(sim.shape, jnp.bool_)
        for _ in range(_TOPK):
            m = jnp.max(s, axis=-1, keepdims=True)
            first = jnp.min(jnp.where(s == m, c, n), axis=-1, keepdims=True)
            sel = c == first
            mask = jnp.logical_or(mask, sel)
            s = jnp.where(sel, _NEG, s)
        mx = jnp.max(jnp.where(mask, sim, _NEG), axis=-1, keepdims=True)
        p = jnp.where(mask, jnp.exp((sim - mx) / _TEMP), 0.0)
        sw = p / jnp.sum(p, axis=-1, keepdims=True)
        s_ref[...] = sw.astype(jnp.bfloat16)
        wgs_ref[...] = jnp.dot(wg1_ref[...], sw,
                               preferred_element_type=jnp.float32).astype(
                                   jnp.bfloat16)

    xb = x_ref[0]

    scores = _dot_t(ws_ref[...], xb) + bs_ref[0, 0]
    m = jnp.max(scores, axis=-1, keepdims=True)
    e = jnp.exp(scores - m)
    aw = e / jnp.sum(e, axis=-1, keepdims=True)
    center_raw = jnp.dot(aw, xb, preferred_element_type=jnp.float32)
    h = _gelu_exact(_dot_t(center_raw, wc1_ref[...]) + bc1_ref[...])
    h = _gelu_exact(_dot_t(h, wc2_ref[...]) + bc2_ref[...])
    cv = _dot_t(h, wcn_ref[...]) + bcn_ref[...]
    cgate = _dot_t(cv, wg2_ref[...]) + bg_ref[...]

    xbh = xb.astype(jnp.bfloat16)
    ring = _dot_t(xbh, s_ref[...])
    gl = _dot_t(xbh, wgs_ref[...]) + cgate
    g = jax.nn.sigmoid(gl)
    fused = cv + g * (ring - cv)
    y = _dot_t(fused.astype(jnp.bfloat16), wfb_ref[...]) + bf_ref[...]
    z = y + xb
    mu = jnp.mean(z, axis=-1, keepdims=True)
    zc = z - mu
    var = jnp.mean(zc * zc, axis=-1, keepdims=True)
    out_ref[0] = zc * jax.lax.rsqrt(var + 1e-5) * lnw_ref[...] + lnb_ref[...]


@jax.jit
def kernel(x, var_embed, Wq, bq, Wk, bk, Ws, bs, Wc1, bc1, Wc2, bc2, Wcn, bcn,
           Wg, bg, Wf, bf, ln_w, ln_b):
    B, L, N = x.shape
    H = var_embed.shape[-1]
    D = Wc1.shape[0]
    f32 = jnp.float32
    bf16 = jnp.bfloat16

    ve = var_embed.reshape(N, H)
    row = lambda v: v.reshape(1, -1)
    Wg2 = Wg[:, N:]
    Wg1 = Wg[:, :N]
    Wfb = Wf.astype(bf16)

    const = lambda *shape: pl.BlockSpec(shape, lambda b: (0,) * len(shape))
    out = pl.pallas_call(
        _fused_kernel,
        grid=(B,),
        in_specs=[
            pl.BlockSpec((1, L, N), lambda b: (b, 0, 0)),
            const(N, H), const(H, H), const(1, H), const(H, H), const(1, H),
            const(1, N), const(1, 1),
            const(D, N), const(1, D), const(D, D), const(1, D),
            const(N, D), const(1, N),
            const(N, N), const(1, N),
            const(N, N), const(N, N), const(1, N), const(1, N), const(1, N),
        ],
        out_specs=pl.BlockSpec((1, L, N), lambda b: (b, 0, 0)),
        out_shape=jax.ShapeDtypeStruct((B, L, N), f32),
        scratch_shapes=[pltpu.VMEM((N, N), bf16), pltpu.VMEM((N, N), bf16)],
    )(x, ve, Wq, row(bq), Wk, row(bk), Ws, bs.reshape(1, 1),
      Wc1, row(bc1), Wc2, row(bc2), Wcn, row(bcn),
      Wg2, row(bg), Wg1, Wfb, row(bf), row(ln_w), row(ln_b))
    return out
